# baseline (speedup 1.0000x reference)
import functools
import jax
import jax.numpy as jnp
from jax import lax
from jax.experimental import pallas as pl
from jax.experimental.pallas import tpu as pltpu
from jax.experimental.pallas import tpu_sc as plsc


def kernel(x, pos_embedding):
    B, S, D = x.shape
    NC, NS = 2, 16
    NW = NC * NS
    SPW = S // NW
    CH = 2
    NCH = SPW // CH
    NBL = 8
    DIST = 6
    NBS = 3
    G = NCH // NBL
    LANES = 16
    HALF = D // (2 * LANES)

    pos = pos_embedding[:S]

    mesh = plsc.VectorSubcoreMesh(core_axis_name="c", subcore_axis_name="s")

    @functools.partial(
        pl.kernel,
        mesh=mesh,
        out_type=jax.ShapeDtypeStruct((B, S, D), jnp.float32),
        scratch_types=[
            pltpu.VMEM((NBL, CH, D), jnp.float32),
            pltpu.VMEM((NBL, B, CH, D), jnp.float32),
            pltpu.VMEM((NBS, B, CH, D), jnp.float32),
        ]
        + [pltpu.SemaphoreType.DMA] * (NBL + NBS),
    )
    def sc_add(x_hbm, pos_hbm, out_hbm, pvb, xvb, svb, *sems):
        lds, sts = sems[:NBL], sems[NBL:]
        w = lax.axis_index("s") * NC + lax.axis_index("c")
        s_w = w * SPW

        def start_load(c, k):
            sb = s_w + c * CH
            pltpu.async_copy(pos_hbm.at[pl.ds(sb, CH)], pvb.at[k], lds[k])
            for b in range(B):
                pltpu.async_copy(x_hbm.at[b, pl.ds(sb, CH)], xvb.at[k, b], lds[k])

        def wait_load(k):
            pltpu.make_async_copy(pos_hbm.at[pl.ds(s_w, CH)], pvb.at[k], lds[k]).wait()
            for b in range(B):
                pltpu.make_async_copy(
                    x_hbm.at[b, pl.ds(s_w, CH)], xvb.at[k, b], lds[k]
                ).wait()

        def start_store(c, m):
            sb = s_w + c * CH
            for b in range(B):
                pltpu.async_copy(svb.at[m, b], out_hbm.at[b, pl.ds(sb, CH)], sts[m])

        def wait_store(m):
            for b in range(B):
                pltpu.make_async_copy(
                    svb.at[m, b], out_hbm.at[b, pl.ds(s_w, CH)], sts[m]
                ).wait()

        def compute(k, m):
            def half_body(hj, carry):
                j = hj >> 1
                h = hj & 1
                pvals = [
                    pvb[k, j, pl.ds((h * HALF + q) * LANES, LANES)]
                    for q in range(HALF)
                ]
                for b in range(B):
                    for q in range(HALF):
                        sl = pl.ds((h * HALF + q) * LANES, LANES)
                        svb[m, b, j, sl] = xvb[k, b, j, sl] + pvals[q]
                return carry

            lax.fori_loop(0, 2 * CH, half_body, 0)

        for c0 in range(DIST):
            start_load(c0, c0)

        def outer_body(g, carry):
            for k in range(NBL):
                c = g * NBL + k
                m = k % NBS
                wait_load(k)
                cp = c + DIST
                kp = (k + DIST) % NBL

                @pl.when(cp < NCH)
                def _prefetch():
                    start_load(cp, kp)

                if k >= NBS:
                    wait_store(m)
                else:

                    @pl.when(g > 0)
                    def _drain():
                        wait_store(m)

                compute(k, m)
                start_store(c, m)

            return carry

        lax.fori_loop(0, G, outer_body, 0)
        for m in range(NBS):
            wait_store(m)

    return sc_add(x, pos)


# speedup vs baseline: 1.3356x; 1.3356x over previous
import functools
import jax
import jax.numpy as jnp
from jax import lax
from jax.experimental import pallas as pl
from jax.experimental.pallas import tpu as pltpu
from jax.experimental.pallas import tpu_sc as plsc


def kernel(x, pos_embedding):
    B, S, D = x.shape
    NC, NS = 2, 16
    NW = NC * NS
    SPW = S // NW
    CH = 4
    NCH = SPW // CH
    NBL = 4
    NBS = 2
    G = NCH // NBL
    LANES = 16
    HALF = D // (2 * LANES)

    pos = pos_embedding[:S]

    mesh = plsc.VectorSubcoreMesh(core_axis_name="c", subcore_axis_name="s")

    @functools.partial(
        pl.kernel,
        mesh=mesh,
        out_type=jax.ShapeDtypeStruct((B, S, D), jnp.float32),
        scratch_types=[
            pltpu.VMEM((NBL, CH, D), jnp.float32),
            pltpu.VMEM((NBL, B, CH, D), jnp.float32),
            pltpu.VMEM((NBS, B, CH, D), jnp.float32),
        ]
        + [pltpu.SemaphoreType.DMA] * (NBL + NBS),
    )
    def sc_add(x_hbm, pos_hbm, out_hbm, pvb, xvb, svb, *sems):
        lds, sts = sems[:NBL], sems[NBL:]
        w = lax.axis_index("s") * NC + lax.axis_index("c")
        s_w = w * SPW

        def start_load(c, k):
            sb = s_w + c * CH
            pltpu.async_copy(pos_hbm.at[pl.ds(sb, CH)], pvb.at[k], lds[k])
            for b in range(B):
                pltpu.async_copy(x_hbm.at[b, pl.ds(sb, CH)], xvb.at[k, b], lds[k])

        def wait_load(k):
            pltpu.make_async_copy(pos_hbm.at[pl.ds(s_w, CH)], pvb.at[k], lds[k]).wait()
            for b in range(B):
                pltpu.make_async_copy(
                    x_hbm.at[b, pl.ds(s_w, CH)], xvb.at[k, b], lds[k]
                ).wait()

        def start_store(c, m):
            sb = s_w + c * CH
            for b in range(B):
                pltpu.async_copy(svb.at[m, b], out_hbm.at[b, pl.ds(sb, CH)], sts[m])

        def wait_store(m):
            for b in range(B):
                pltpu.make_async_copy(
                    svb.at[m, b], out_hbm.at[b, pl.ds(s_w, CH)], sts[m]
                ).wait()

        def compute(k, m):
            def row_body(j, carry):
                for h in range(2):
                    pvals = [
                        pvb[k, j, pl.ds((h * HALF + q) * LANES, LANES)]
                        for q in range(HALF)
                    ]
                    for b in range(B):
                        for q in range(HALF):
                            sl = pl.ds((h * HALF + q) * LANES, LANES)
                            svb[m, b, j, sl] = xvb[k, b, j, sl] + pvals[q]
                return carry

            lax.fori_loop(0, CH, row_body, 0)

        for c0 in range(NBL - 1):
            start_load(c0, c0)

        def outer_body(g, carry):
            for k in range(NBL):
                c = g * NBL + k
                m = k % NBS
                wait_load(k)
                cp = c + NBL - 1
                kp = (k + NBL - 1) % NBL

                @pl.when(cp < NCH)
                def _prefetch():
                    start_load(cp, kp)

                if k >= NBS:
                    wait_store(m)
                else:

                    @pl.when(g > 0)
                    def _drain():
                        wait_store(m)

                compute(k, m)
                start_store(c, m)

            return carry

        lax.fori_loop(0, G, outer_body, 0)
        for m in range(NBS):
            wait_store(m)

    return sc_add(x, pos)
